# baseline (speedup 1.0000x reference)
import functools

import jax
import jax.numpy as jnp
from jax import lax
from jax.experimental import pallas as pl
from jax.experimental.pallas import tpu as pltpu
from jax.experimental.pallas import tpu_sc as plsc

NC = 2
NS = 16
NW = NC * NS
L = 16
D = 256
P = 25
PPAD = 32


def _body(b_per_w, n_chunk, ch,
          patterns_hbm, scale_hbm, shift_hbm, idx_hbm, out_hbm,
          pat_v, sc_v, sh_v, tab_v, idx_v, st0, st1, p0, p1):
    w = lax.axis_index("s") * NC + lax.axis_index("c")
    base = w * b_per_w
    pltpu.sync_copy(patterns_hbm, pat_v)
    pltpu.sync_copy(scale_hbm, sc_v)
    pltpu.sync_copy(shift_hbm, sh_v)
    pltpu.sync_copy(idx_hbm.at[pl.ds(base, b_per_w)], idx_v)

    def build_row(p, _):
        pv = jnp.full((L,), p, jnp.int32)
        scale = plsc.load_gather(sc_v, [pv])
        shf = plsc.load_gather(sh_v, [pv])
        s = (shf * 10.0).astype(jnp.int32)
        for c in range(D // L):
            col = lax.iota(jnp.int32, L) + (c * L)
            src = lax.rem(col - s, D)
            src = src + jnp.where(src < 0, D, 0)
            vals = plsc.load_gather(pat_v, [pv, src])
            tab_v[pl.ds(p * D + c * L, L)] = vals * scale
        return 0

    lax.fori_loop(0, P, build_row, 0)

    sts = (st0, st1)
    ps = (p0, p1)

    def construct(c, st):
        @plsc.parallel_loop(0, ch // L, unroll=2)
        def group(g):
            ivec = idx_v[pl.ds(c * ch + g * L, L)]
            rbs = [ivec[k] * D for k in range(L)]
            for q in range(D // L):
                for k in range(L):
                    st[g * L + k, pl.ds(q * L, L)] = (
                        tab_v[pl.ds(rbs[k] + q * L, L)])

    def put(c, b):
        return pltpu.make_async_copy(
            sts[b], out_hbm.at[pl.ds(base + c * ch, ch)], ps[b])

    construct(0, st0)
    put(0, 0).start()
    construct(1, st1)
    put(1, 1).start()

    def body(i, _):
        c = 2 * i + 2
        put(c - 2, 0).wait()
        construct(c, st0)
        put(c, 0).start()
        put(c - 1, 1).wait()
        construct(c + 1, st1)
        put(c + 1, 1).start()
        return 0

    lax.fori_loop(0, (n_chunk - 2) // 2, body, 0)
    put(n_chunk - 2, 0).wait()
    put(n_chunk - 1, 1).wait()


def kernel(indices, patterns, amplitude_scale, frequency_shift):
    bsz, seq = indices.shape
    n = bsz * seq
    b_per_w = n // NW
    ch = 128
    n_chunk = b_per_w // ch

    mesh = plsc.VectorSubcoreMesh(
        core_axis_name="c", subcore_axis_name="s",
        num_cores=NC, num_subcores=NS)

    scale_p = jnp.zeros((PPAD,), jnp.float32).at[:P].set(amplitude_scale)
    shift_p = jnp.zeros((PPAD,), jnp.float32).at[:P].set(frequency_shift)

    run = pl.kernel(
        functools.partial(_body, b_per_w, n_chunk, ch),
        out_type=jax.ShapeDtypeStruct((n, D), jnp.float32),
        mesh=mesh,
        compiler_params=pltpu.CompilerParams(needs_layout_passes=False),
        scratch_types=[
            pltpu.VMEM((P, D), jnp.float32),
            pltpu.VMEM((PPAD,), jnp.float32),
            pltpu.VMEM((PPAD,), jnp.float32),
            pltpu.VMEM((P * D,), jnp.float32),
            pltpu.VMEM((b_per_w,), jnp.int32),
            pltpu.VMEM((ch, D), jnp.float32),
            pltpu.VMEM((ch, D), jnp.float32),
            pltpu.SemaphoreType.DMA,
            pltpu.SemaphoreType.DMA,
        ],
    )
    out = run(patterns, scale_p, shift_p, indices.reshape(n))
    return out.reshape(bsz, seq, D)


# speedup vs baseline: 1.8213x; 1.8213x over previous
import functools

import jax
import jax.numpy as jnp
from jax import lax
from jax.experimental import pallas as pl
from jax.experimental.pallas import tpu as pltpu
from jax.experimental.pallas import tpu_sc as plsc

NC = 2
NS = 16
NW = NC * NS
L = 16
D = 256
P = 25
PPAD = 32
K = 64


def _fold_body(patterns_hbm, scale_hbm, shift_hbm, folded_hbm,
               pat_v, sc_v, sh_v, out_v):
    w = lax.axis_index("s") * NC + lax.axis_index("c")
    pltpu.sync_copy(patterns_hbm, pat_v)
    pltpu.sync_copy(scale_hbm, sc_v)
    pltpu.sync_copy(shift_hbm, sh_v)

    @pl.when(w < P)
    def _():
        wv = jnp.full((L,), w, jnp.int32)
        scale = plsc.load_gather(sc_v, [wv])
        shf = plsc.load_gather(sh_v, [wv])
        s = (shf * 10.0).astype(jnp.int32)
        for c in range(D // L):
            col = lax.iota(jnp.int32, L) + (c * L)
            src = lax.rem(col - s, D)
            src = src + jnp.where(src < 0, D, 0)
            vals = plsc.load_gather(pat_v, [wv, src])
            out_v[pl.ds(c * L, L)] = vals * scale
        for k in range(K):
            pltpu.sync_copy(out_v, folded_hbm.at[k * P + w])


def _gather_body(b_per_w, n_chunk, ch,
                 folded_hbm, idx_hbm, out_hbm,
                 idx_v, rows0, rows1, g0, g1, p0, p1):
    w = lax.axis_index("s") * NC + lax.axis_index("c")
    base = w * b_per_w
    pltpu.sync_copy(idx_hbm.at[pl.ds(base, b_per_w)], idx_v)

    def spread(j, _):
        offs = ((lax.iota(jnp.int32, L) + j * L) % K) * P
        idx_v[pl.ds(j * L, L)] = idx_v[pl.ds(j * L, L)] + offs
        return 0

    lax.fori_loop(0, b_per_w // L, spread, 0)
    rows = (rows0, rows1)
    gs = (g0, g1)
    ps = (p0, p1)

    def gath(c, b):
        return pltpu.make_async_copy(
            folded_hbm.at[idx_v.at[pl.ds(c * ch, ch)]], rows[b], gs[b])

    def put(c, b):
        return pltpu.make_async_copy(
            rows[b], out_hbm.at[pl.ds(base + c * ch, ch)], ps[b])

    gath(0, 0).start()
    gath(1, 1).start()

    def body(i, _):
        c0 = 2 * i
        for b in range(2):
            gath(c0 + b, b).wait()
            put(c0 + b, b).start()
        for b in range(2):
            put(c0 + b, b).wait()

            @pl.when(c0 + b + 2 < n_chunk)
            def _():
                gath(c0 + b + 2, b).start()
        return 0

    lax.fori_loop(0, n_chunk // 2, body, 0)


def kernel(indices, patterns, amplitude_scale, frequency_shift):
    bsz, seq = indices.shape
    n = bsz * seq
    b_per_w = n // NW
    ch = 128
    n_chunk = b_per_w // ch

    mesh = plsc.VectorSubcoreMesh(
        core_axis_name="c", subcore_axis_name="s",
        num_cores=NC, num_subcores=NS)

    scale_p = jnp.zeros((PPAD,), jnp.float32).at[:P].set(amplitude_scale)
    shift_p = jnp.zeros((PPAD,), jnp.float32).at[:P].set(frequency_shift)

    fold = pl.kernel(
        _fold_body,
        out_type=jax.ShapeDtypeStruct((K * P, D), jnp.float32),
        mesh=mesh,
        compiler_params=pltpu.CompilerParams(needs_layout_passes=False),
        scratch_types=[
            pltpu.VMEM((P, D), jnp.float32),
            pltpu.VMEM((PPAD,), jnp.float32),
            pltpu.VMEM((PPAD,), jnp.float32),
            pltpu.VMEM((D,), jnp.float32),
        ],
    )
    folded = fold(patterns, scale_p, shift_p)

    gather = pl.kernel(
        functools.partial(_gather_body, b_per_w, n_chunk, ch),
        out_type=jax.ShapeDtypeStruct((n, D), jnp.float32),
        mesh=mesh,
        compiler_params=pltpu.CompilerParams(needs_layout_passes=False),
        scratch_types=[
            pltpu.VMEM((b_per_w,), jnp.int32),
            pltpu.VMEM((ch, D), jnp.float32),
            pltpu.VMEM((ch, D), jnp.float32),
            pltpu.SemaphoreType.DMA,
            pltpu.SemaphoreType.DMA,
            pltpu.SemaphoreType.DMA,
            pltpu.SemaphoreType.DMA,
        ],
    )
    out = gather(folded, indices.reshape(n))
    return out.reshape(bsz, seq, D)


# speedup vs baseline: 1.8292x; 1.0043x over previous
import functools

import jax
import jax.numpy as jnp
from jax import lax
from jax.experimental import pallas as pl
from jax.experimental.pallas import tpu as pltpu
from jax.experimental.pallas import tpu_sc as plsc

NC = 2
NS = 16
NW = NC * NS
L = 16
D = 256
P = 25
PPAD = 32
K = 64


def _fold_body(patterns_hbm, scale_hbm, shift_hbm, folded_hbm,
               pat_v, sc_v, sh_v, out_v):
    w = lax.axis_index("s") * NC + lax.axis_index("c")
    pltpu.sync_copy(patterns_hbm, pat_v)
    pltpu.sync_copy(scale_hbm, sc_v)
    pltpu.sync_copy(shift_hbm, sh_v)

    @pl.when(w < P)
    def _():
        wv = jnp.full((L,), w, jnp.int32)
        scale = plsc.load_gather(sc_v, [wv])
        shf = plsc.load_gather(sh_v, [wv])
        s = (shf * 10.0).astype(jnp.int32)
        for c in range(D // L):
            col = lax.iota(jnp.int32, L) + (c * L)
            src = lax.rem(col - s, D)
            src = src + jnp.where(src < 0, D, 0)
            vals = plsc.load_gather(pat_v, [wv, src])
            out_v[pl.ds(c * L, L)] = vals * scale
        for k in range(K):
            pltpu.sync_copy(out_v, folded_hbm.at[k * P + w])


def _gather_body(b_per_w, n_chunk, ch,
                 folded_hbm, idx_hbm, out_hbm,
                 idx_v, rows0, rows1, g0, g1, p0, p1):
    w = lax.axis_index("s") * NC + lax.axis_index("c")
    base = w * b_per_w
    pltpu.sync_copy(idx_hbm.at[pl.ds(base, b_per_w)], idx_v)

    def spread(j, _):
        offs = ((lax.iota(jnp.int32, L) + j * L) % K) * P
        idx_v[pl.ds(j * L, L)] = idx_v[pl.ds(j * L, L)] + offs
        return 0

    lax.fori_loop(0, b_per_w // L, spread, 0)
    rows = (rows0, rows1)
    gs = (g0, g1)
    ps = (p0, p1)

    def gath(c, b):
        return pltpu.make_async_copy(
            folded_hbm.at[idx_v.at[pl.ds(c * ch, ch)]], rows[b], gs[b])

    def put(c, b):
        return pltpu.make_async_copy(
            rows[b], out_hbm.at[pl.ds(base + c * ch, ch)], ps[b])

    gath(0, 0).start()
    gath(1, 1).start()

    def body(i, _):
        c0 = 2 * i
        for b in range(2):
            gath(c0 + b, b).wait()
            put(c0 + b, b).start()
        for b in range(2):
            put(c0 + b, b).wait()

            @pl.when(c0 + b + 2 < n_chunk)
            def _():
                gath(c0 + b + 2, b).start()
        return 0

    lax.fori_loop(0, n_chunk // 2, body, 0)


def kernel(indices, patterns, amplitude_scale, frequency_shift):
    bsz, seq = indices.shape
    n = bsz * seq
    b_per_w = n // NW
    ch = 200
    n_chunk = b_per_w // ch

    mesh = plsc.VectorSubcoreMesh(
        core_axis_name="c", subcore_axis_name="s",
        num_cores=NC, num_subcores=NS)

    scale_p = jnp.zeros((PPAD,), jnp.float32).at[:P].set(amplitude_scale)
    shift_p = jnp.zeros((PPAD,), jnp.float32).at[:P].set(frequency_shift)

    fold = pl.kernel(
        _fold_body,
        out_type=jax.ShapeDtypeStruct((K * P, D), jnp.float32),
        mesh=mesh,
        compiler_params=pltpu.CompilerParams(needs_layout_passes=False),
        scratch_types=[
            pltpu.VMEM((P, D), jnp.float32),
            pltpu.VMEM((PPAD,), jnp.float32),
            pltpu.VMEM((PPAD,), jnp.float32),
            pltpu.VMEM((D,), jnp.float32),
        ],
    )
    folded = fold(patterns, scale_p, shift_p)

    gather = pl.kernel(
        functools.partial(_gather_body, b_per_w, n_chunk, ch),
        out_type=jax.ShapeDtypeStruct((n, D), jnp.float32),
        mesh=mesh,
        compiler_params=pltpu.CompilerParams(needs_layout_passes=False),
        scratch_types=[
            pltpu.VMEM((b_per_w,), jnp.int32),
            pltpu.VMEM((ch, D), jnp.float32),
            pltpu.VMEM((ch, D), jnp.float32),
            pltpu.SemaphoreType.DMA,
            pltpu.SemaphoreType.DMA,
            pltpu.SemaphoreType.DMA,
            pltpu.SemaphoreType.DMA,
        ],
    )
    out = gather(folded, indices.reshape(n))
    return out.reshape(bsz, seq, D)


# speedup vs baseline: 1.8874x; 1.0318x over previous
import functools

import jax
import jax.numpy as jnp
from jax import lax
from jax.experimental import pallas as pl
from jax.experimental.pallas import tpu as pltpu
from jax.experimental.pallas import tpu_sc as plsc

NC = 2
NS = 16
NW = NC * NS
L = 16
D = 256
P = 25
PPAD = 32
KH = 2 * NS


def _body(b_per_w, n_chunk, ch,
          patterns_hbm, scale_hbm, shift_hbm, idx_hbm,
          out_hbm, rep_hbm,
          pat_v, sc_v, sh_v, tab_v, idx_v, rows0, rows1,
          gi, g0, g1, p0, p1):
    cid = lax.axis_index("c")
    sid = lax.axis_index("s")
    w = sid * NC + cid
    base = w * b_per_w

    idx_cp = pltpu.make_async_copy(
        idx_hbm.at[pl.ds(base, b_per_w)], idx_v, gi)
    idx_cp.start()
    pltpu.sync_copy(patterns_hbm, pat_v)
    pltpu.sync_copy(scale_hbm, sc_v)
    pltpu.sync_copy(shift_hbm, sh_v)

    def build_row(p, _):
        pv = jnp.full((L,), p, jnp.int32)
        scale = plsc.load_gather(sc_v, [pv])
        shf = plsc.load_gather(sh_v, [pv])
        s = (shf * 10.0).astype(jnp.int32)
        for c in range(D // L):
            col = lax.iota(jnp.int32, L) + (c * L)
            src = lax.rem(col - s, D)
            src = src + jnp.where(src < 0, D, 0)
            vals = plsc.load_gather(pat_v, [pv, src])
            tab_v[p, pl.ds(c * L, L)] = vals * scale
        return 0

    lax.fori_loop(0, P, build_row, 0)

    r0 = (cid * KH + 2 * sid) * PPAD
    pltpu.sync_copy(tab_v, rep_hbm.at[pl.ds(r0, PPAD)])
    pltpu.sync_copy(tab_v, rep_hbm.at[pl.ds(r0 + PPAD, PPAD)])

    idx_cp.wait()

    def spread(j, _):
        offs = (cid * KH + ((lax.iota(jnp.int32, L) + j * L) % KH)) * PPAD
        idx_v[pl.ds(j * L, L)] = idx_v[pl.ds(j * L, L)] + offs
        return 0

    lax.fori_loop(0, b_per_w // L, spread, 0)
    plsc.subcore_barrier()

    rows = (rows0, rows1)
    gs = (g0, g1)
    ps = (p0, p1)

    def gath(c, b):
        return pltpu.make_async_copy(
            rep_hbm.at[idx_v.at[pl.ds(c * ch, ch)]], rows[b], gs[b])

    def put(c, b):
        return pltpu.make_async_copy(
            rows[b], out_hbm.at[pl.ds(base + c * ch, ch)], ps[b])

    gath(0, 0).start()
    gath(1, 1).start()

    def body(i, _):
        c0 = 2 * i
        for b in range(2):
            gath(c0 + b, b).wait()
            put(c0 + b, b).start()
        for b in range(2):
            put(c0 + b, b).wait()

            @pl.when(c0 + b + 2 < n_chunk)
            def _():
                gath(c0 + b + 2, b).start()
        return 0

    lax.fori_loop(0, n_chunk // 2, body, 0)


def kernel(indices, patterns, amplitude_scale, frequency_shift):
    bsz, seq = indices.shape
    n = bsz * seq
    b_per_w = n // NW
    ch = 200
    n_chunk = b_per_w // ch

    mesh = plsc.VectorSubcoreMesh(
        core_axis_name="c", subcore_axis_name="s",
        num_cores=NC, num_subcores=NS)

    scale_p = jnp.zeros((PPAD,), jnp.float32).at[:P].set(amplitude_scale)
    shift_p = jnp.zeros((PPAD,), jnp.float32).at[:P].set(frequency_shift)

    run = pl.kernel(
        functools.partial(_body, b_per_w, n_chunk, ch),
        out_type=(
            jax.ShapeDtypeStruct((n, D), jnp.float32),
            jax.ShapeDtypeStruct((NC * KH * PPAD, D), jnp.float32),
        ),
        mesh=mesh,
        compiler_params=pltpu.CompilerParams(needs_layout_passes=False),
        scratch_types=[
            pltpu.VMEM((P, D), jnp.float32),
            pltpu.VMEM((PPAD,), jnp.float32),
            pltpu.VMEM((PPAD,), jnp.float32),
            pltpu.VMEM((PPAD, D), jnp.float32),
            pltpu.VMEM((b_per_w,), jnp.int32),
            pltpu.VMEM((ch, D), jnp.float32),
            pltpu.VMEM((ch, D), jnp.float32),
            pltpu.SemaphoreType.DMA,
            pltpu.SemaphoreType.DMA,
            pltpu.SemaphoreType.DMA,
            pltpu.SemaphoreType.DMA,
            pltpu.SemaphoreType.DMA,
        ],
    )
    out, _ = run(patterns, scale_p, shift_p, indices.reshape(n))
    return out.reshape(bsz, seq, D)


# speedup vs baseline: 2.0081x; 1.0640x over previous
import functools

import jax
import jax.numpy as jnp
from jax import lax
from jax.experimental import pallas as pl
from jax.experimental.pallas import tpu as pltpu
from jax.experimental.pallas import tpu_sc as plsc

NC = 2
NS = 16
NW = NC * NS
L = 16
D = 256
P = 25
PPAD = 32
KR = 4
KH = KR * NS


def _body(b_per_w, n_chunk, ch,
          patterns_hbm, scale_hbm, shift_hbm, idx_hbm,
          out_hbm, rep_hbm,
          pat_v, sc_v, sh_v, tab_v, idx_v, rows0, rows1,
          gi, g0, g1, p0, p1):
    cid = lax.axis_index("c")
    sid = lax.axis_index("s")
    w = sid * NC + cid
    base = w * b_per_w

    idx_cp = pltpu.make_async_copy(
        idx_hbm.at[pl.ds(base, b_per_w)], idx_v, gi)
    idx_cp.start()
    pltpu.sync_copy(patterns_hbm, pat_v)
    pltpu.sync_copy(scale_hbm, sc_v)
    pltpu.sync_copy(shift_hbm, sh_v)

    def build_row(p, _):
        pv = jnp.full((L,), p, jnp.int32)
        scale = plsc.load_gather(sc_v, [pv])
        shf = plsc.load_gather(sh_v, [pv])
        s = (shf * 10.0).astype(jnp.int32)
        for c in range(D // L):
            col = lax.iota(jnp.int32, L) + (c * L)
            src = lax.rem(col - s, D)
            src = src + jnp.where(src < 0, D, 0)
            vals = plsc.load_gather(pat_v, [pv, src])
            tab_v[p, pl.ds(c * L, L)] = vals * scale
        return 0

    lax.fori_loop(0, P, build_row, 0)

    r0 = (cid * KH + KR * sid) * PPAD
    for k in range(KR):
        pltpu.sync_copy(tab_v, rep_hbm.at[pl.ds(r0 + k * PPAD, PPAD)])

    idx_cp.wait()

    def spread(j, _):
        offs = (cid * KH + ((lax.iota(jnp.int32, L) + j * L) % KH)) * PPAD
        idx_v[pl.ds(j * L, L)] = idx_v[pl.ds(j * L, L)] + offs
        return 0

    lax.fori_loop(0, b_per_w // L, spread, 0)
    plsc.subcore_barrier()

    rows = (rows0, rows1)
    gs = (g0, g1)
    ps = (p0, p1)

    def gath(c, b):
        return pltpu.make_async_copy(
            rep_hbm.at[idx_v.at[pl.ds(c * ch, ch)]], rows[b], gs[b])

    def put(c, b):
        return pltpu.make_async_copy(
            rows[b], out_hbm.at[pl.ds(base + c * ch, ch)], ps[b])

    gath(0, 0).start()
    gath(1, 1).start()

    def body(i, _):
        c0 = 2 * i
        for b in range(2):
            gath(c0 + b, b).wait()
            put(c0 + b, b).start()
        for b in range(2):
            put(c0 + b, b).wait()

            @pl.when(c0 + b + 2 < n_chunk)
            def _():
                gath(c0 + b + 2, b).start()
        return 0

    lax.fori_loop(0, n_chunk // 2, body, 0)


def kernel(indices, patterns, amplitude_scale, frequency_shift):
    bsz, seq = indices.shape
    n = bsz * seq
    b_per_w = n // NW
    ch = 200
    n_chunk = b_per_w // ch

    mesh = plsc.VectorSubcoreMesh(
        core_axis_name="c", subcore_axis_name="s",
        num_cores=NC, num_subcores=NS)

    scale_p = jnp.zeros((PPAD,), jnp.float32).at[:P].set(amplitude_scale)
    shift_p = jnp.zeros((PPAD,), jnp.float32).at[:P].set(frequency_shift)

    run = pl.kernel(
        functools.partial(_body, b_per_w, n_chunk, ch),
        out_type=(
            jax.ShapeDtypeStruct((n, D), jnp.float32),
            jax.ShapeDtypeStruct((NC * KH * PPAD, D), jnp.float32),
        ),
        mesh=mesh,
        compiler_params=pltpu.CompilerParams(needs_layout_passes=False),
        scratch_types=[
            pltpu.VMEM((P, D), jnp.float32),
            pltpu.VMEM((PPAD,), jnp.float32),
            pltpu.VMEM((PPAD,), jnp.float32),
            pltpu.VMEM((PPAD, D), jnp.float32),
            pltpu.VMEM((b_per_w,), jnp.int32),
            pltpu.VMEM((ch, D), jnp.float32),
            pltpu.VMEM((ch, D), jnp.float32),
            pltpu.SemaphoreType.DMA,
            pltpu.SemaphoreType.DMA,
            pltpu.SemaphoreType.DMA,
            pltpu.SemaphoreType.DMA,
            pltpu.SemaphoreType.DMA,
        ],
    )
    out, _ = run(patterns, scale_p, shift_p, indices.reshape(n))
    return out.reshape(bsz, seq, D)


# speedup vs baseline: 2.0758x; 1.0337x over previous
import functools

import jax
import jax.numpy as jnp
from jax import lax
from jax.experimental import pallas as pl
from jax.experimental.pallas import tpu as pltpu
from jax.experimental.pallas import tpu_sc as plsc

NC = 2
NS = 16
NW = NC * NS
L = 16
D = 256
P = 25
PPAD = 32
KR = 8
KH = KR * NS


def _body(b_per_w, n_chunk, ch,
          patterns_hbm, scale_hbm, shift_hbm, idx_hbm,
          out_hbm, rep_hbm,
          pat_v, sc_v, sh_v, tab_v, idx_v, rows0, rows1,
          gi, g0, g1, p0, p1):
    cid = lax.axis_index("c")
    sid = lax.axis_index("s")
    w = sid * NC + cid
    base = w * b_per_w

    idx_cp = pltpu.make_async_copy(
        idx_hbm.at[pl.ds(base, b_per_w)], idx_v, gi)
    idx_cp.start()
    pltpu.sync_copy(patterns_hbm, pat_v)
    pltpu.sync_copy(scale_hbm, sc_v)
    pltpu.sync_copy(shift_hbm, sh_v)

    def build_row(p, _):
        pv = jnp.full((L,), p, jnp.int32)
        scale = plsc.load_gather(sc_v, [pv])
        shf = plsc.load_gather(sh_v, [pv])
        s = (shf * 10.0).astype(jnp.int32)
        for c in range(D // L):
            col = lax.iota(jnp.int32, L) + (c * L)
            src = lax.rem(col - s, D)
            src = src + jnp.where(src < 0, D, 0)
            vals = plsc.load_gather(pat_v, [pv, src])
            tab_v[p, pl.ds(c * L, L)] = vals * scale
        return 0

    lax.fori_loop(0, P, build_row, 0)

    r0 = (cid * KH + KR * sid) * PPAD
    for k in range(KR):
        pltpu.sync_copy(tab_v, rep_hbm.at[pl.ds(r0 + k * PPAD, PPAD)])

    idx_cp.wait()

    def spread(j, _):
        offs = (cid * KH + ((lax.iota(jnp.int32, L) + j * L) % KH)) * PPAD
        idx_v[pl.ds(j * L, L)] = idx_v[pl.ds(j * L, L)] + offs
        return 0

    lax.fori_loop(0, b_per_w // L, spread, 0)
    plsc.subcore_barrier()

    rows = (rows0, rows1)
    gs = (g0, g1)
    ps = (p0, p1)

    def gath(c, b):
        return pltpu.make_async_copy(
            rep_hbm.at[idx_v.at[pl.ds(c * ch, ch)]], rows[b], gs[b])

    def put(c, b):
        return pltpu.make_async_copy(
            rows[b], out_hbm.at[pl.ds(base + c * ch, ch)], ps[b])

    gath(0, 0).start()
    gath(1, 1).start()

    def body(i, _):
        c0 = 2 * i
        for b in range(2):
            gath(c0 + b, b).wait()
            put(c0 + b, b).start()
        for b in range(2):
            put(c0 + b, b).wait()

            @pl.when(c0 + b + 2 < n_chunk)
            def _():
                gath(c0 + b + 2, b).start()
        return 0

    lax.fori_loop(0, n_chunk // 2, body, 0)


def kernel(indices, patterns, amplitude_scale, frequency_shift):
    bsz, seq = indices.shape
    n = bsz * seq
    b_per_w = n // NW
    ch = 200
    n_chunk = b_per_w // ch

    mesh = plsc.VectorSubcoreMesh(
        core_axis_name="c", subcore_axis_name="s",
        num_cores=NC, num_subcores=NS)

    scale_p = jnp.zeros((PPAD,), jnp.float32).at[:P].set(amplitude_scale)
    shift_p = jnp.zeros((PPAD,), jnp.float32).at[:P].set(frequency_shift)

    run = pl.kernel(
        functools.partial(_body, b_per_w, n_chunk, ch),
        out_type=(
            jax.ShapeDtypeStruct((n, D), jnp.float32),
            jax.ShapeDtypeStruct((NC * KH * PPAD, D), jnp.float32),
        ),
        mesh=mesh,
        compiler_params=pltpu.CompilerParams(needs_layout_passes=False),
        scratch_types=[
            pltpu.VMEM((P, D), jnp.float32),
            pltpu.VMEM((PPAD,), jnp.float32),
            pltpu.VMEM((PPAD,), jnp.float32),
            pltpu.VMEM((PPAD, D), jnp.float32),
            pltpu.VMEM((b_per_w,), jnp.int32),
            pltpu.VMEM((ch, D), jnp.float32),
            pltpu.VMEM((ch, D), jnp.float32),
            pltpu.SemaphoreType.DMA,
            pltpu.SemaphoreType.DMA,
            pltpu.SemaphoreType.DMA,
            pltpu.SemaphoreType.DMA,
            pltpu.SemaphoreType.DMA,
        ],
    )
    out, _ = run(patterns, scale_p, shift_p, indices.reshape(n))
    return out.reshape(bsz, seq, D)


# speedup vs baseline: 2.0784x; 1.0012x over previous
import functools

import jax
import jax.numpy as jnp
from jax import lax
from jax.experimental import pallas as pl
from jax.experimental.pallas import tpu as pltpu
from jax.experimental.pallas import tpu_sc as plsc

NC = 2
NS = 16
NW = NC * NS
L = 16
D = 256
P = 25
PPAD = 32
KR = 16
KH = KR * NS


def _body(b_per_w, n_chunk, ch,
          patterns_hbm, scale_hbm, shift_hbm, idx_hbm,
          out_hbm, rep_hbm,
          pat_v, sc_v, sh_v, tab_v, idx_v, rows0, rows1,
          gi, g0, g1, p0, p1):
    cid = lax.axis_index("c")
    sid = lax.axis_index("s")
    w = sid * NC + cid
    base = w * b_per_w

    idx_cp = pltpu.make_async_copy(
        idx_hbm.at[pl.ds(base, b_per_w)], idx_v, gi)
    idx_cp.start()
    pltpu.sync_copy(patterns_hbm, pat_v)
    pltpu.sync_copy(scale_hbm, sc_v)
    pltpu.sync_copy(shift_hbm, sh_v)

    def build_row(p, _):
        pv = jnp.full((L,), p, jnp.int32)
        scale = plsc.load_gather(sc_v, [pv])
        shf = plsc.load_gather(sh_v, [pv])
        s = (shf * 10.0).astype(jnp.int32)
        for c in range(D // L):
            col = lax.iota(jnp.int32, L) + (c * L)
            src = lax.rem(col - s, D)
            src = src + jnp.where(src < 0, D, 0)
            vals = plsc.load_gather(pat_v, [pv, src])
            tab_v[p, pl.ds(c * L, L)] = vals * scale
        return 0

    lax.fori_loop(0, P, build_row, 0)

    r0 = (cid * KH + KR * sid) * PPAD
    for k in range(KR):
        pltpu.sync_copy(tab_v, rep_hbm.at[pl.ds(r0 + k * PPAD, PPAD)])

    idx_cp.wait()

    def spread(j, _):
        offs = (cid * KH + ((lax.iota(jnp.int32, L) + j * L) % KH)) * PPAD
        idx_v[pl.ds(j * L, L)] = idx_v[pl.ds(j * L, L)] + offs
        return 0

    lax.fori_loop(0, b_per_w // L, spread, 0)
    plsc.subcore_barrier()

    rows = (rows0, rows1)
    gs = (g0, g1)
    ps = (p0, p1)

    def gath(c, b):
        return pltpu.make_async_copy(
            rep_hbm.at[idx_v.at[pl.ds(c * ch, ch)]], rows[b], gs[b])

    def put(c, b):
        return pltpu.make_async_copy(
            rows[b], out_hbm.at[pl.ds(base + c * ch, ch)], ps[b])

    gath(0, 0).start()
    gath(1, 1).start()

    def body(i, _):
        c0 = 2 * i
        for b in range(2):
            gath(c0 + b, b).wait()
            put(c0 + b, b).start()
        for b in range(2):
            put(c0 + b, b).wait()

            @pl.when(c0 + b + 2 < n_chunk)
            def _():
                gath(c0 + b + 2, b).start()
        return 0

    lax.fori_loop(0, n_chunk // 2, body, 0)


def kernel(indices, patterns, amplitude_scale, frequency_shift):
    bsz, seq = indices.shape
    n = bsz * seq
    b_per_w = n // NW
    ch = 200
    n_chunk = b_per_w // ch

    mesh = plsc.VectorSubcoreMesh(
        core_axis_name="c", subcore_axis_name="s",
        num_cores=NC, num_subcores=NS)

    scale_p = jnp.zeros((PPAD,), jnp.float32).at[:P].set(amplitude_scale)
    shift_p = jnp.zeros((PPAD,), jnp.float32).at[:P].set(frequency_shift)

    run = pl.kernel(
        functools.partial(_body, b_per_w, n_chunk, ch),
        out_type=(
            jax.ShapeDtypeStruct((n, D), jnp.float32),
            jax.ShapeDtypeStruct((NC * KH * PPAD, D), jnp.float32),
        ),
        mesh=mesh,
        compiler_params=pltpu.CompilerParams(needs_layout_passes=False),
        scratch_types=[
            pltpu.VMEM((P, D), jnp.float32),
            pltpu.VMEM((PPAD,), jnp.float32),
            pltpu.VMEM((PPAD,), jnp.float32),
            pltpu.VMEM((PPAD, D), jnp.float32),
            pltpu.VMEM((b_per_w,), jnp.int32),
            pltpu.VMEM((ch, D), jnp.float32),
            pltpu.VMEM((ch, D), jnp.float32),
            pltpu.SemaphoreType.DMA,
            pltpu.SemaphoreType.DMA,
            pltpu.SemaphoreType.DMA,
            pltpu.SemaphoreType.DMA,
            pltpu.SemaphoreType.DMA,
        ],
    )
    out, _ = run(patterns, scale_p, shift_p, indices.reshape(n))
    return out.reshape(bsz, seq, D)
